# baseline (speedup 1.0000x reference)
import functools

import jax
import jax.numpy as jnp
from jax.experimental import pallas as pl
from jax.experimental.pallas import tpu as pltpu

_VMEM_LIMIT = 100 * 1024 * 1024


def _lane_shift(a, s, zeros):
    if s == 0:
        return a
    if s > 0:
        return jnp.concatenate([a[..., s:], zeros[..., :s]], axis=-1)
    return jnp.concatenate([zeros[..., :(-s)], a[..., :s]], axis=-1)


def _conv_stats_kernel(x_ref, w_ref, yt_ref, psum_ref, psq_ref, *, kk, ho, wo):
    xs = x_ref[...].astype(jnp.bfloat16)
    nb, cin, hw = xs.shape
    pad = (kk - 1) // 2
    zeros = jnp.zeros_like(xs)
    col = jax.lax.broadcasted_iota(jnp.int32, (1, 1, hw), 2) % wo
    taps = []
    for kh in range(kk):
        dh = kh - pad
        for kw in range(kk):
            dw = kw - pad
            t = _lane_shift(xs, dh * wo + dw, zeros)
            if dw < 0:
                t = jnp.where(col >= -dw, t, 0).astype(jnp.bfloat16)
            elif dw > 0:
                t = jnp.where(col < wo - dw, t, 0).astype(jnp.bfloat16)
            taps.append(t)
    pt = jnp.concatenate(taps, axis=1)
    pt2 = jnp.concatenate([pt[j] for j in range(nb)], axis=-1)
    wt = w_ref[...]
    yf = jnp.dot(wt, pt2, preferred_element_type=jnp.float32)
    for j in range(nb):
        yt_ref[j] = yf[:, j * hw:(j + 1) * hw].astype(yt_ref.dtype)
    psum_ref[...] = jnp.sum(yf, axis=1, keepdims=True)[None]
    psq_ref[...] = jnp.sum(yf * yf, axis=1, keepdims=True)[None]


def _bn_relu_kernel(y_ref, scale_ref, shift_ref, o_ref):
    o_ref[...] = jnp.maximum(
        y_ref[...].astype(jnp.float32) * scale_ref[...] + shift_ref[...], 0.0
    )


@functools.partial(jax.jit, static_argnames=())
def kernel(x, w, b, gamma, beta):
    eps = 1e-5
    N, Cin, H, W = x.shape
    Cout = w.shape[0]
    K = w.shape[2]
    Ho, Wo = H, W
    HW = Ho * Wo
    M = N * HW
    KKC = K * K * Cin
    del b

    x3 = x.reshape(N, Cin, HW)
    w2dt = jnp.transpose(w, (0, 2, 3, 1)).reshape(Cout, KKC).astype(jnp.bfloat16)

    nb = 2 if N % 2 == 0 else 1
    G = N // nb
    body = functools.partial(_conv_stats_kernel, kk=K, ho=Ho, wo=Wo)
    yt, psum, psq = pl.pallas_call(
        body,
        out_shape=(
            jax.ShapeDtypeStruct((N, Cout, HW), jnp.bfloat16),
            jax.ShapeDtypeStruct((G, Cout, 1), jnp.float32),
            jax.ShapeDtypeStruct((G, Cout, 1), jnp.float32),
        ),
        grid=(G,),
        in_specs=[
            pl.BlockSpec((nb, Cin, HW), lambda i: (i, 0, 0)),
            pl.BlockSpec((Cout, KKC), lambda i: (0, 0)),
        ],
        out_specs=[
            pl.BlockSpec((nb, Cout, HW), lambda i: (i, 0, 0)),
            pl.BlockSpec((1, Cout, 1), lambda i: (i, 0, 0)),
            pl.BlockSpec((1, Cout, 1), lambda i: (i, 0, 0)),
        ],
        compiler_params=pltpu.CompilerParams(
            dimension_semantics=("parallel",),
            vmem_limit_bytes=_VMEM_LIMIT,
        ),
        cost_estimate=pl.CostEstimate(
            flops=2 * M * KKC * Cout,
            transcendentals=0,
            bytes_accessed=4 * M * Cin + 2 * KKC * Cout + 2 * M * Cout,
        ),
    )(x3, w2dt)

    inv_m = 1.0 / float(M)
    mean = jnp.sum(psum, axis=0) * inv_m
    var = jnp.maximum(jnp.sum(psq, axis=0) * inv_m - mean * mean, 0.0)
    g2d = gamma.reshape(Cout, 1).astype(jnp.float32)
    b2d = beta.reshape(Cout, 1).astype(jnp.float32)
    scale = g2d * jax.lax.rsqrt(var + eps)
    shift = b2d - mean * scale
    nb2 = 8
    while N % nb2:
        nb2 //= 2
    scale_b = jnp.broadcast_to(scale[None], (nb2, Cout, HW)).reshape(nb2 * Cout, HW)
    shift_b = jnp.broadcast_to(shift[None], (nb2, Cout, HW)).reshape(nb2 * Cout, HW)

    y2 = yt.reshape(N * Cout, HW)
    out2 = pl.pallas_call(
        _bn_relu_kernel,
        out_shape=jax.ShapeDtypeStruct((N * Cout, HW), jnp.float32),
        grid=(N // nb2,),
        in_specs=[
            pl.BlockSpec((nb2 * Cout, HW), lambda i: (i, 0)),
            pl.BlockSpec((nb2 * Cout, HW), lambda i: (0, 0)),
            pl.BlockSpec((nb2 * Cout, HW), lambda i: (0, 0)),
        ],
        out_specs=pl.BlockSpec((nb2 * Cout, HW), lambda i: (i, 0)),
        compiler_params=pltpu.CompilerParams(
            dimension_semantics=("parallel",),
            vmem_limit_bytes=_VMEM_LIMIT,
        ),
        cost_estimate=pl.CostEstimate(
            flops=3 * M * Cout,
            transcendentals=0,
            bytes_accessed=6 * M * Cout,
        ),
    )(y2, scale_b, shift_b)

    return out2.reshape(N, Cout, Ho, Wo)


# speedup vs baseline: 1.5365x; 1.5365x over previous
import functools

import jax
import jax.numpy as jnp
from jax.experimental import pallas as pl
from jax.experimental.pallas import tpu as pltpu

_VMEM_LIMIT = 100 * 1024 * 1024


def _to_nhwc_pad_kernel(x_ref, eye_ref, o_ref, *, ho, wo, pad):
    xs = x_ref[...].astype(jnp.bfloat16)
    nb0, cin, _ = xs.shape
    eye = eye_ref[...]
    outs = []
    for j in range(nb0):
        xt = jax.lax.dot_general(
            xs[j], eye,
            dimension_numbers=(((0,), (0,)), ((), ())),
            preferred_element_type=jnp.float32,
        )
        outs.append(xt.astype(jnp.bfloat16).reshape(ho, wo, cin))
    o_ref[...] = jnp.pad(
        jnp.stack(outs),
        ((0, 0), (pad, pad), (pad, pad), (0, 0)),
    )


def _conv_stats_kernel(x_ref, w_ref, y_ref, psum_ref, psq_ref, *, kk, ho, wo):
    xs = x_ref[...]
    nb = xs.shape[0]
    cols = [
        xs[:, kh:kh + ho, kw:kw + wo, :]
        for kh in range(kk) for kw in range(kk)
    ]
    p = jnp.concatenate(cols, axis=-1).reshape(nb * ho * wo, -1)
    yf = jnp.dot(p, w_ref[...], preferred_element_type=jnp.float32)
    y_ref[...] = yf.astype(y_ref.dtype)
    psum_ref[...] = jnp.sum(yf, axis=0, keepdims=True)[None]
    psq_ref[...] = jnp.sum(yf * yf, axis=0, keepdims=True)[None]


def _bn_relu_kernel(y_ref, scale_ref, shift_ref, o_ref):
    o_ref[...] = jnp.maximum(
        y_ref[...].astype(jnp.float32) * scale_ref[...] + shift_ref[...], 0.0
    )


@functools.partial(jax.jit, static_argnames=())
def kernel(x, w, b, gamma, beta):
    eps = 1e-5
    N, Cin, H, W = x.shape
    Cout = w.shape[0]
    K = w.shape[2]
    Ho, Wo = H, W
    HW = Ho * Wo
    M = N * HW
    KKC = K * K * Cin
    pad = (K - 1) // 2
    Hp, Wp = Ho + 2 * pad, Wo + 2 * pad
    del b

    x3 = x.reshape(N, Cin, HW)
    w2d = jnp.transpose(w, (2, 3, 1, 0)).reshape(KKC, Cout).astype(jnp.bfloat16)
    eye = jnp.eye(Cin, dtype=jnp.bfloat16)

    nb0 = 2 if N % 2 == 0 else 1
    body0 = functools.partial(_to_nhwc_pad_kernel, ho=Ho, wo=Wo, pad=pad)
    x_sp = pl.pallas_call(
        body0,
        out_shape=jax.ShapeDtypeStruct((N, Hp, Wp, Cin), jnp.bfloat16),
        grid=(N // nb0,),
        in_specs=[
            pl.BlockSpec((nb0, Cin, HW), lambda i: (i, 0, 0)),
            pl.BlockSpec((Cin, Cin), lambda i: (0, 0)),
        ],
        out_specs=pl.BlockSpec((nb0, Hp, Wp, Cin), lambda i: (i, 0, 0, 0)),
        compiler_params=pltpu.CompilerParams(
            dimension_semantics=("parallel",),
            vmem_limit_bytes=_VMEM_LIMIT,
        ),
        cost_estimate=pl.CostEstimate(
            flops=2 * M * Cin * Cin,
            transcendentals=0,
            bytes_accessed=4 * M * Cin + 2 * N * Hp * Wp * Cin,
        ),
    )(x3, eye)

    nb = 2 if N % 2 == 0 else 1
    G = N // nb
    body = functools.partial(_conv_stats_kernel, kk=K, ho=Ho, wo=Wo)
    y2d, psum, psq = pl.pallas_call(
        body,
        out_shape=(
            jax.ShapeDtypeStruct((M, Cout), jnp.bfloat16),
            jax.ShapeDtypeStruct((G, 1, Cout), jnp.float32),
            jax.ShapeDtypeStruct((G, 1, Cout), jnp.float32),
        ),
        grid=(G,),
        in_specs=[
            pl.BlockSpec((nb, Hp, Wp, Cin), lambda i: (i, 0, 0, 0)),
            pl.BlockSpec((KKC, Cout), lambda i: (0, 0)),
        ],
        out_specs=[
            pl.BlockSpec((nb * HW, Cout), lambda i: (i, 0)),
            pl.BlockSpec((1, 1, Cout), lambda i: (i, 0, 0)),
            pl.BlockSpec((1, 1, Cout), lambda i: (i, 0, 0)),
        ],
        compiler_params=pltpu.CompilerParams(
            dimension_semantics=("parallel",),
            vmem_limit_bytes=_VMEM_LIMIT,
        ),
        cost_estimate=pl.CostEstimate(
            flops=2 * M * KKC * Cout,
            transcendentals=0,
            bytes_accessed=2 * N * Hp * Wp * Cin + 2 * KKC * Cout + 2 * M * Cout,
        ),
    )(x_sp, w2d)

    inv_m = 1.0 / float(M)
    mean = jnp.sum(psum, axis=0) * inv_m
    var = jnp.maximum(jnp.sum(psq, axis=0) * inv_m - mean * mean, 0.0)
    g2d = gamma.reshape(1, Cout).astype(jnp.float32)
    b2d = beta.reshape(1, Cout).astype(jnp.float32)
    scale = g2d * jax.lax.rsqrt(var + eps)
    shift = b2d - mean * scale

    tm = 4096
    while M % tm:
        tm //= 2
    out2d = pl.pallas_call(
        _bn_relu_kernel,
        out_shape=jax.ShapeDtypeStruct((M, Cout), jnp.float32),
        grid=(M // tm,),
        in_specs=[
            pl.BlockSpec((tm, Cout), lambda i: (i, 0)),
            pl.BlockSpec((1, Cout), lambda i: (0, 0)),
            pl.BlockSpec((1, Cout), lambda i: (0, 0)),
        ],
        out_specs=pl.BlockSpec((tm, Cout), lambda i: (i, 0)),
        compiler_params=pltpu.CompilerParams(
            dimension_semantics=("parallel",),
            vmem_limit_bytes=_VMEM_LIMIT,
        ),
        cost_estimate=pl.CostEstimate(
            flops=3 * M * Cout,
            transcendentals=0,
            bytes_accessed=6 * M * Cout,
        ),
    )(y2d, scale, shift)

    return jnp.transpose(out2d.reshape(N, Ho, Wo, Cout), (0, 3, 1, 2))


# speedup vs baseline: 2.1377x; 1.3912x over previous
import functools

import jax
import jax.numpy as jnp
from jax.experimental import pallas as pl
from jax.experimental.pallas import tpu as pltpu

_VMEM_LIMIT = 100 * 1024 * 1024


def _conv_stats_kernel(x_ref, w_ref, y_ref, psum_ref, psq_ref, *, kk, ho, wo):
    xs = x_ref[...]
    nb = xs.shape[0]
    cols = [
        xs[:, kh:kh + ho, kw:kw + wo, :]
        for kh in range(kk) for kw in range(kk)
    ]
    p = jnp.concatenate(cols, axis=-1).reshape(nb * ho * wo, -1)
    yf = jnp.dot(p, w_ref[...], preferred_element_type=jnp.float32)
    y_ref[...] = yf.astype(y_ref.dtype)
    psum_ref[...] = jnp.sum(yf, axis=0, keepdims=True)[None]
    psq_ref[...] = jnp.sum(yf * yf, axis=0, keepdims=True)[None]


def _bn_relu_kernel(y_ref, scale_ref, shift_ref, o_ref):
    o_ref[...] = jnp.maximum(
        y_ref[...].astype(jnp.float32) * scale_ref[...] + shift_ref[...], 0.0
    )


@functools.partial(jax.jit, static_argnames=())
def kernel(x, w, b, gamma, beta):
    eps = 1e-5
    N, Cin, H, W = x.shape
    Cout = w.shape[0]
    K = w.shape[2]
    Ho, Wo = H, W
    HW = Ho * Wo
    M = N * HW
    KKC = K * K * Cin
    pad = (K - 1) // 2
    Hp, Wp = Ho + 2 * pad, Wo + 2 * pad
    del b

    x_sp = jnp.pad(
        jnp.transpose(x, (0, 2, 3, 1)),
        ((0, 0), (pad, pad), (pad, pad), (0, 0)),
    ).astype(jnp.bfloat16)
    w2d = jnp.transpose(w, (2, 3, 1, 0)).reshape(KKC, Cout).astype(jnp.bfloat16)

    nb = 2 if N % 2 == 0 else 1
    G = N // nb
    body = functools.partial(_conv_stats_kernel, kk=K, ho=Ho, wo=Wo)
    y2d, psum, psq = pl.pallas_call(
        body,
        out_shape=(
            jax.ShapeDtypeStruct((M, Cout), jnp.bfloat16),
            jax.ShapeDtypeStruct((G, 1, Cout), jnp.float32),
            jax.ShapeDtypeStruct((G, 1, Cout), jnp.float32),
        ),
        grid=(G,),
        in_specs=[
            pl.BlockSpec((nb, Hp, Wp, Cin), lambda i: (i, 0, 0, 0)),
            pl.BlockSpec((KKC, Cout), lambda i: (0, 0)),
        ],
        out_specs=[
            pl.BlockSpec((nb * HW, Cout), lambda i: (i, 0)),
            pl.BlockSpec((1, 1, Cout), lambda i: (i, 0, 0)),
            pl.BlockSpec((1, 1, Cout), lambda i: (i, 0, 0)),
        ],
        compiler_params=pltpu.CompilerParams(
            dimension_semantics=("parallel",),
            vmem_limit_bytes=_VMEM_LIMIT,
        ),
        cost_estimate=pl.CostEstimate(
            flops=2 * M * KKC * Cout,
            transcendentals=0,
            bytes_accessed=2 * N * Hp * Wp * Cin + 2 * KKC * Cout + 2 * M * Cout,
        ),
    )(x_sp, w2d)

    inv_m = 1.0 / float(M)
    mean = jnp.sum(psum, axis=0) * inv_m
    var = jnp.maximum(jnp.sum(psq, axis=0) * inv_m - mean * mean, 0.0)
    g2d = gamma.reshape(1, Cout).astype(jnp.float32)
    b2d = beta.reshape(1, Cout).astype(jnp.float32)
    scale = g2d * jax.lax.rsqrt(var + eps)
    shift = b2d - mean * scale

    tm = 4096
    while M % tm:
        tm //= 2
    out2d = pl.pallas_call(
        _bn_relu_kernel,
        out_shape=jax.ShapeDtypeStruct((M, Cout), jnp.float32),
        grid=(M // tm,),
        in_specs=[
            pl.BlockSpec((tm, Cout), lambda i: (i, 0)),
            pl.BlockSpec((1, Cout), lambda i: (0, 0)),
            pl.BlockSpec((1, Cout), lambda i: (0, 0)),
        ],
        out_specs=pl.BlockSpec((tm, Cout), lambda i: (i, 0)),
        compiler_params=pltpu.CompilerParams(
            dimension_semantics=("parallel",),
            vmem_limit_bytes=_VMEM_LIMIT,
        ),
        cost_estimate=pl.CostEstimate(
            flops=3 * M * Cout,
            transcendentals=0,
            bytes_accessed=6 * M * Cout,
        ),
    )(y2d, scale, shift)

    return jnp.transpose(out2d.reshape(N, Ho, Wo, Cout), (0, 3, 1, 2))


# speedup vs baseline: 2.2228x; 1.0398x over previous
import functools

import jax
import jax.numpy as jnp
from jax.experimental import pallas as pl
from jax.experimental.pallas import tpu as pltpu

_VMEM_LIMIT = 100 * 1024 * 1024


def _conv_stats_kernel(x_ref, w_ref, y_ref, psum_ref, psq_ref, *, kk, ho, wo):
    xs = x_ref[...]
    nb = xs.shape[0]
    cols = [
        xs[:, kh:kh + ho, kw:kw + wo, :]
        for kh in range(kk) for kw in range(kk)
    ]
    p = jnp.concatenate(cols, axis=-1).reshape(nb * ho * wo, -1)
    yf = jnp.dot(p, w_ref[...], preferred_element_type=jnp.float32)
    y_ref[...] = yf.astype(y_ref.dtype)
    psum_ref[...] = jnp.sum(yf, axis=0, keepdims=True)[None]
    psq_ref[...] = jnp.sum(yf * yf, axis=0, keepdims=True)[None]


def _bn_relu_kernel(y_ref, scale_ref, shift_ref, o_ref):
    o_ref[...] = jnp.maximum(
        y_ref[...].astype(jnp.float32) * scale_ref[...] + shift_ref[...], 0.0
    )


@functools.partial(jax.jit, static_argnames=())
def kernel(x, w, b, gamma, beta):
    eps = 1e-5
    N, Cin, H, W = x.shape
    Cout = w.shape[0]
    K = w.shape[2]
    Ho, Wo = H, W
    HW = Ho * Wo
    M = N * HW
    KKC = K * K * Cin
    pad = (K - 1) // 2
    Hp, Wp = Ho + 2 * pad, Wo + 2 * pad
    del b

    x_sp = jnp.pad(
        jnp.transpose(x, (0, 2, 3, 1)),
        ((0, 0), (pad, pad), (pad, pad), (0, 0)),
    ).astype(jnp.bfloat16)
    w2d = jnp.transpose(w, (2, 3, 1, 0)).reshape(KKC, Cout).astype(jnp.bfloat16)

    nb = 4
    while N % nb:
        nb //= 2
    G = N // nb
    body = functools.partial(_conv_stats_kernel, kk=K, ho=Ho, wo=Wo)
    y2d, psum, psq = pl.pallas_call(
        body,
        out_shape=(
            jax.ShapeDtypeStruct((M, Cout), jnp.bfloat16),
            jax.ShapeDtypeStruct((G, 1, Cout), jnp.float32),
            jax.ShapeDtypeStruct((G, 1, Cout), jnp.float32),
        ),
        grid=(G,),
        in_specs=[
            pl.BlockSpec((nb, Hp, Wp, Cin), lambda i: (i, 0, 0, 0)),
            pl.BlockSpec((KKC, Cout), lambda i: (0, 0)),
        ],
        out_specs=[
            pl.BlockSpec((nb * HW, Cout), lambda i: (i, 0)),
            pl.BlockSpec((1, 1, Cout), lambda i: (i, 0, 0)),
            pl.BlockSpec((1, 1, Cout), lambda i: (i, 0, 0)),
        ],
        compiler_params=pltpu.CompilerParams(
            dimension_semantics=("parallel",),
            vmem_limit_bytes=_VMEM_LIMIT,
        ),
        cost_estimate=pl.CostEstimate(
            flops=2 * M * KKC * Cout,
            transcendentals=0,
            bytes_accessed=2 * N * Hp * Wp * Cin + 2 * KKC * Cout + 2 * M * Cout,
        ),
    )(x_sp, w2d)

    inv_m = 1.0 / float(M)
    mean = jnp.sum(psum, axis=0) * inv_m
    var = jnp.maximum(jnp.sum(psq, axis=0) * inv_m - mean * mean, 0.0)
    g2d = gamma.reshape(1, Cout).astype(jnp.float32)
    b2d = beta.reshape(1, Cout).astype(jnp.float32)
    scale = g2d * jax.lax.rsqrt(var + eps)
    shift = b2d - mean * scale

    tm = 4096
    while M % tm:
        tm //= 2
    out2d = pl.pallas_call(
        _bn_relu_kernel,
        out_shape=jax.ShapeDtypeStruct((M, Cout), jnp.float32),
        grid=(M // tm,),
        in_specs=[
            pl.BlockSpec((tm, Cout), lambda i: (i, 0)),
            pl.BlockSpec((1, Cout), lambda i: (0, 0)),
            pl.BlockSpec((1, Cout), lambda i: (0, 0)),
        ],
        out_specs=pl.BlockSpec((tm, Cout), lambda i: (i, 0)),
        compiler_params=pltpu.CompilerParams(
            dimension_semantics=("parallel",),
            vmem_limit_bytes=_VMEM_LIMIT,
        ),
        cost_estimate=pl.CostEstimate(
            flops=3 * M * Cout,
            transcendentals=0,
            bytes_accessed=6 * M * Cout,
        ),
    )(y2d, scale, shift)

    return jnp.transpose(out2d.reshape(N, Ho, Wo, Cout), (0, 3, 1, 2))


# speedup vs baseline: 2.2527x; 1.0134x over previous
import functools

import jax
import jax.numpy as jnp
from jax.experimental import pallas as pl
from jax.experimental.pallas import tpu as pltpu

_VMEM_LIMIT = 100 * 1024 * 1024


def _conv_stats_kernel(x_ref, w_ref, y_ref, psum_ref, psq_ref, *, kk, ho, wo):
    xs = x_ref[...]
    nb = xs.shape[0]
    cols = [
        xs[:, kh:kh + ho, kw:kw + wo, :]
        for kh in range(kk) for kw in range(kk)
    ]
    p = jnp.concatenate(cols, axis=-1).reshape(nb * ho * wo, -1)
    yf = jnp.dot(p, w_ref[...], preferred_element_type=jnp.float32)
    y_ref[...] = yf.astype(y_ref.dtype)
    psum_ref[...] = jnp.sum(yf, axis=0, keepdims=True)[None]
    psq_ref[...] = jnp.sum(yf * yf, axis=0, keepdims=True)[None]


def _bn_relu_kernel(y_ref, scale_ref, shift_ref, o_ref):
    o_ref[...] = jnp.maximum(
        y_ref[...].astype(jnp.float32) * scale_ref[...] + shift_ref[...], 0.0
    )


@functools.partial(jax.jit, static_argnames=())
def kernel(x, w, b, gamma, beta):
    eps = 1e-5
    N, Cin, H, W = x.shape
    Cout = w.shape[0]
    K = w.shape[2]
    Ho, Wo = H, W
    HW = Ho * Wo
    M = N * HW
    KKC = K * K * Cin
    pad = (K - 1) // 2
    Hp, Wp = Ho + 2 * pad, Wo + 2 * pad
    del b

    x_sp = jnp.pad(
        jnp.transpose(x, (0, 2, 3, 1)),
        ((0, 0), (pad, pad), (pad, pad), (0, 0)),
    ).astype(jnp.bfloat16)
    w2d = jnp.transpose(w, (2, 3, 1, 0)).reshape(KKC, Cout).astype(jnp.bfloat16)

    nb = 8
    while N % nb:
        nb //= 2
    G = N // nb
    body = functools.partial(_conv_stats_kernel, kk=K, ho=Ho, wo=Wo)
    y2d, psum, psq = pl.pallas_call(
        body,
        out_shape=(
            jax.ShapeDtypeStruct((M, Cout), jnp.bfloat16),
            jax.ShapeDtypeStruct((G, 1, Cout), jnp.float32),
            jax.ShapeDtypeStruct((G, 1, Cout), jnp.float32),
        ),
        grid=(G,),
        in_specs=[
            pl.BlockSpec((nb, Hp, Wp, Cin), lambda i: (i, 0, 0, 0)),
            pl.BlockSpec((KKC, Cout), lambda i: (0, 0)),
        ],
        out_specs=[
            pl.BlockSpec((nb * HW, Cout), lambda i: (i, 0)),
            pl.BlockSpec((1, 1, Cout), lambda i: (i, 0, 0)),
            pl.BlockSpec((1, 1, Cout), lambda i: (i, 0, 0)),
        ],
        compiler_params=pltpu.CompilerParams(
            dimension_semantics=("parallel",),
            vmem_limit_bytes=_VMEM_LIMIT,
        ),
        cost_estimate=pl.CostEstimate(
            flops=2 * M * KKC * Cout,
            transcendentals=0,
            bytes_accessed=2 * N * Hp * Wp * Cin + 2 * KKC * Cout + 2 * M * Cout,
        ),
    )(x_sp, w2d)

    inv_m = 1.0 / float(M)
    mean = jnp.sum(psum, axis=0) * inv_m
    var = jnp.maximum(jnp.sum(psq, axis=0) * inv_m - mean * mean, 0.0)
    g2d = gamma.reshape(1, Cout).astype(jnp.float32)
    b2d = beta.reshape(1, Cout).astype(jnp.float32)
    scale = g2d * jax.lax.rsqrt(var + eps)
    shift = b2d - mean * scale

    tm = 4096
    while M % tm:
        tm //= 2
    out2d = pl.pallas_call(
        _bn_relu_kernel,
        out_shape=jax.ShapeDtypeStruct((M, Cout), jnp.float32),
        grid=(M // tm,),
        in_specs=[
            pl.BlockSpec((tm, Cout), lambda i: (i, 0)),
            pl.BlockSpec((1, Cout), lambda i: (0, 0)),
            pl.BlockSpec((1, Cout), lambda i: (0, 0)),
        ],
        out_specs=pl.BlockSpec((tm, Cout), lambda i: (i, 0)),
        compiler_params=pltpu.CompilerParams(
            dimension_semantics=("parallel",),
            vmem_limit_bytes=_VMEM_LIMIT,
        ),
        cost_estimate=pl.CostEstimate(
            flops=3 * M * Cout,
            transcendentals=0,
            bytes_accessed=6 * M * Cout,
        ),
    )(y2d, scale, shift)

    return jnp.transpose(out2d.reshape(N, Ho, Wo, Cout), (0, 3, 1, 2))


# speedup vs baseline: 2.4310x; 1.0792x over previous
import functools

import jax
import jax.numpy as jnp
from jax.experimental import pallas as pl
from jax.experimental.pallas import tpu as pltpu

_VMEM_LIMIT = 100 * 1024 * 1024


def _conv_stats_kernel(x_ref, w_ref, y_ref, psum_ref, psq_ref, *, kk, ho, wo):
    pad = (kk - 1) // 2
    xs = jnp.pad(
        x_ref[...],
        ((0, 0), (pad, pad), (pad, pad), (0, 0)),
    )
    nb = xs.shape[0]
    cols = [
        xs[:, kh:kh + ho, kw:kw + wo, :]
        for kh in range(kk) for kw in range(kk)
    ]
    p = jnp.concatenate(cols, axis=-1).reshape(nb * ho * wo, -1)
    yf = jnp.dot(p, w_ref[...], preferred_element_type=jnp.float32)
    y_ref[...] = yf.astype(y_ref.dtype)
    psum_ref[...] = jnp.sum(yf, axis=0, keepdims=True)[None]
    psq_ref[...] = jnp.sum(yf * yf, axis=0, keepdims=True)[None]


def _bn_relu_kernel(y_ref, scale_ref, shift_ref, o_ref):
    o_ref[...] = jnp.maximum(
        y_ref[...].astype(jnp.float32) * scale_ref[...] + shift_ref[...], 0.0
    )


@functools.partial(jax.jit, static_argnames=())
def kernel(x, w, b, gamma, beta):
    eps = 1e-5
    N, Cin, H, W = x.shape
    Cout = w.shape[0]
    K = w.shape[2]
    Ho, Wo = H, W
    HW = Ho * Wo
    M = N * HW
    KKC = K * K * Cin
    pad = (K - 1) // 2
    Hp, Wp = Ho + 2 * pad, Wo + 2 * pad
    del b

    x_nhwc = jnp.transpose(x, (0, 2, 3, 1)).astype(jnp.bfloat16)
    w2d = jnp.transpose(w, (2, 3, 1, 0)).reshape(KKC, Cout).astype(jnp.bfloat16)

    nb = 8
    while N % nb:
        nb //= 2
    G = N // nb
    body = functools.partial(_conv_stats_kernel, kk=K, ho=Ho, wo=Wo)
    y2d, psum, psq = pl.pallas_call(
        body,
        out_shape=(
            jax.ShapeDtypeStruct((M, Cout), jnp.bfloat16),
            jax.ShapeDtypeStruct((G, 1, Cout), jnp.float32),
            jax.ShapeDtypeStruct((G, 1, Cout), jnp.float32),
        ),
        grid=(G,),
        in_specs=[
            pl.BlockSpec((nb, Ho, Wo, Cin), lambda i: (i, 0, 0, 0)),
            pl.BlockSpec((KKC, Cout), lambda i: (0, 0)),
        ],
        out_specs=[
            pl.BlockSpec((nb * HW, Cout), lambda i: (i, 0)),
            pl.BlockSpec((1, 1, Cout), lambda i: (i, 0, 0)),
            pl.BlockSpec((1, 1, Cout), lambda i: (i, 0, 0)),
        ],
        compiler_params=pltpu.CompilerParams(
            dimension_semantics=("parallel",),
            vmem_limit_bytes=_VMEM_LIMIT,
        ),
        cost_estimate=pl.CostEstimate(
            flops=2 * M * KKC * Cout,
            transcendentals=0,
            bytes_accessed=2 * M * Cin + 2 * KKC * Cout + 2 * M * Cout,
        ),
    )(x_nhwc, w2d)

    inv_m = 1.0 / float(M)
    mean = jnp.sum(psum, axis=0) * inv_m
    var = jnp.maximum(jnp.sum(psq, axis=0) * inv_m - mean * mean, 0.0)
    g2d = gamma.reshape(1, Cout).astype(jnp.float32)
    b2d = beta.reshape(1, Cout).astype(jnp.float32)
    scale = g2d * jax.lax.rsqrt(var + eps)
    shift = b2d - mean * scale

    tm = 4096
    while M % tm:
        tm //= 2
    out2d = pl.pallas_call(
        _bn_relu_kernel,
        out_shape=jax.ShapeDtypeStruct((M, Cout), jnp.float32),
        grid=(M // tm,),
        in_specs=[
            pl.BlockSpec((tm, Cout), lambda i: (i, 0)),
            pl.BlockSpec((1, Cout), lambda i: (0, 0)),
            pl.BlockSpec((1, Cout), lambda i: (0, 0)),
        ],
        out_specs=pl.BlockSpec((tm, Cout), lambda i: (i, 0)),
        compiler_params=pltpu.CompilerParams(
            dimension_semantics=("parallel",),
            vmem_limit_bytes=_VMEM_LIMIT,
        ),
        cost_estimate=pl.CostEstimate(
            flops=3 * M * Cout,
            transcendentals=0,
            bytes_accessed=6 * M * Cout,
        ),
    )(y2d, scale, shift)

    return jnp.transpose(out2d.reshape(N, Ho, Wo, Cout), (0, 3, 1, 2))


# speedup vs baseline: 3.0238x; 1.2439x over previous
import functools

import jax
import jax.numpy as jnp
from jax.experimental import pallas as pl
from jax.experimental.pallas import tpu as pltpu

_VMEM_LIMIT = 100 * 1024 * 1024


def _conv_stats_kernel(x_ref, w_ref, y_ref, psum_ref, psq_ref, *, kk, ho, wo):
    pad = (kk - 1) // 2
    xs = jnp.pad(
        x_ref[...].astype(jnp.bfloat16),
        ((0, 0), (pad, pad), (pad, pad), (0, 0)),
    )
    nb = xs.shape[0]
    cols = [
        xs[:, kh:kh + ho, kw:kw + wo, :]
        for kh in range(kk) for kw in range(kk)
    ]
    p = jnp.concatenate(cols, axis=-1).reshape(nb * ho * wo, -1)
    yf = jnp.dot(p, w_ref[...], preferred_element_type=jnp.float32)
    y_ref[...] = yf.astype(y_ref.dtype)
    psum_ref[...] = jnp.sum(yf, axis=0, keepdims=True)[None]
    psq_ref[...] = jnp.sum(yf * yf, axis=0, keepdims=True)[None]


def _bn_relu_kernel(y_ref, scale_ref, shift_ref, o_ref):
    o_ref[...] = jnp.maximum(
        y_ref[...].astype(jnp.float32) * scale_ref[...] + shift_ref[...], 0.0
    )


@functools.partial(jax.jit, static_argnames=())
def kernel(x, w, b, gamma, beta):
    eps = 1e-5
    N, Cin, H, W = x.shape
    Cout = w.shape[0]
    K = w.shape[2]
    Ho, Wo = H, W
    HW = Ho * Wo
    M = N * HW
    KKC = K * K * Cin
    pad = (K - 1) // 2
    Hp, Wp = Ho + 2 * pad, Wo + 2 * pad
    del b

    x_nhwc = jnp.transpose(x, (0, 2, 3, 1))
    w2d = jnp.transpose(w, (2, 3, 1, 0)).reshape(KKC, Cout).astype(jnp.bfloat16)

    nb = 8
    while N % nb:
        nb //= 2
    G = N // nb
    body = functools.partial(_conv_stats_kernel, kk=K, ho=Ho, wo=Wo)
    y2d, psum, psq = pl.pallas_call(
        body,
        out_shape=(
            jax.ShapeDtypeStruct((M, Cout), jnp.bfloat16),
            jax.ShapeDtypeStruct((G, 1, Cout), jnp.float32),
            jax.ShapeDtypeStruct((G, 1, Cout), jnp.float32),
        ),
        grid=(G,),
        in_specs=[
            pl.BlockSpec((nb, Ho, Wo, Cin), lambda i: (i, 0, 0, 0)),
            pl.BlockSpec((KKC, Cout), lambda i: (0, 0)),
        ],
        out_specs=[
            pl.BlockSpec((nb * HW, Cout), lambda i: (i, 0)),
            pl.BlockSpec((1, 1, Cout), lambda i: (i, 0, 0)),
            pl.BlockSpec((1, 1, Cout), lambda i: (i, 0, 0)),
        ],
        compiler_params=pltpu.CompilerParams(
            dimension_semantics=("parallel",),
            vmem_limit_bytes=_VMEM_LIMIT,
        ),
        cost_estimate=pl.CostEstimate(
            flops=2 * M * KKC * Cout,
            transcendentals=0,
            bytes_accessed=2 * M * Cin + 2 * KKC * Cout + 2 * M * Cout,
        ),
    )(x_nhwc, w2d)

    inv_m = 1.0 / float(M)
    mean = jnp.sum(psum, axis=0) * inv_m
    var = jnp.maximum(jnp.sum(psq, axis=0) * inv_m - mean * mean, 0.0)
    g2d = gamma.reshape(1, Cout).astype(jnp.float32)
    b2d = beta.reshape(1, Cout).astype(jnp.float32)
    scale = g2d * jax.lax.rsqrt(var + eps)
    shift = b2d - mean * scale

    tm = 4096
    while M % tm:
        tm //= 2
    out2d = pl.pallas_call(
        _bn_relu_kernel,
        out_shape=jax.ShapeDtypeStruct((M, Cout), jnp.float32),
        grid=(M // tm,),
        in_specs=[
            pl.BlockSpec((tm, Cout), lambda i: (i, 0)),
            pl.BlockSpec((1, Cout), lambda i: (0, 0)),
            pl.BlockSpec((1, Cout), lambda i: (0, 0)),
        ],
        out_specs=pl.BlockSpec((tm, Cout), lambda i: (i, 0)),
        compiler_params=pltpu.CompilerParams(
            dimension_semantics=("parallel",),
            vmem_limit_bytes=_VMEM_LIMIT,
        ),
        cost_estimate=pl.CostEstimate(
            flops=3 * M * Cout,
            transcendentals=0,
            bytes_accessed=6 * M * Cout,
        ),
    )(y2d, scale, shift)

    return jnp.transpose(out2d.reshape(N, Ho, Wo, Cout), (0, 3, 1, 2))


# speedup vs baseline: 3.1878x; 1.0543x over previous
import functools

import jax
import jax.numpy as jnp
from jax.experimental import pallas as pl
from jax.experimental.pallas import tpu as pltpu

_VMEM_LIMIT = 100 * 1024 * 1024


def _conv_stats_kernel(x_ref, w_ref, y_ref, psum_ref, psq_ref, *, kk, ho, wo):
    pad = (kk - 1) // 2
    xs = jnp.pad(
        x_ref[...].astype(jnp.bfloat16),
        ((0, 0), (pad, pad), (pad, pad), (0, 0)),
    )
    nb = xs.shape[0]
    cols = [
        xs[:, kh:kh + ho, kw:kw + wo, :]
        for kh in range(kk) for kw in range(kk)
    ]
    p = jnp.concatenate(cols, axis=-1).reshape(nb * ho * wo, -1)
    yf = jnp.dot(p, w_ref[...], preferred_element_type=jnp.float32)
    y_ref[...] = yf.astype(y_ref.dtype)
    psum_ref[...] = jnp.sum(yf, axis=0, keepdims=True)[None]
    psq_ref[...] = jnp.sum(yf * yf, axis=0, keepdims=True)[None]


def _bn_relu_kernel(y_ref, scale_ref, shift_ref, o_ref):
    o_ref[...] = jnp.maximum(
        y_ref[...].astype(jnp.float32) * scale_ref[...] + shift_ref[...], 0.0
    )


@functools.partial(jax.jit, static_argnames=())
def kernel(x, w, b, gamma, beta):
    eps = 1e-5
    N, Cin, H, W = x.shape
    Cout = w.shape[0]
    K = w.shape[2]
    Ho, Wo = H, W
    HW = Ho * Wo
    M = N * HW
    KKC = K * K * Cin
    pad = (K - 1) // 2
    Hp, Wp = Ho + 2 * pad, Wo + 2 * pad
    del b

    x_nhwc = jnp.transpose(x, (0, 2, 3, 1))
    w2d = jnp.transpose(w, (2, 3, 1, 0)).reshape(KKC, Cout).astype(jnp.bfloat16)

    nb = 8
    while N % nb:
        nb //= 2
    G = N // nb
    body = functools.partial(_conv_stats_kernel, kk=K, ho=Ho, wo=Wo)
    y2d, psum, psq = pl.pallas_call(
        body,
        out_shape=(
            jax.ShapeDtypeStruct((M, Cout), jnp.bfloat16),
            jax.ShapeDtypeStruct((G, 1, Cout), jnp.float32),
            jax.ShapeDtypeStruct((G, 1, Cout), jnp.float32),
        ),
        grid=(G,),
        in_specs=[
            pl.BlockSpec((nb, Ho, Wo, Cin), lambda i: (i, 0, 0, 0)),
            pl.BlockSpec((KKC, Cout), lambda i: (0, 0)),
        ],
        out_specs=[
            pl.BlockSpec((nb * HW, Cout), lambda i: (i, 0)),
            pl.BlockSpec((1, 1, Cout), lambda i: (i, 0, 0)),
            pl.BlockSpec((1, 1, Cout), lambda i: (i, 0, 0)),
        ],
        compiler_params=pltpu.CompilerParams(
            dimension_semantics=("parallel",),
            vmem_limit_bytes=_VMEM_LIMIT,
        ),
        cost_estimate=pl.CostEstimate(
            flops=2 * M * KKC * Cout,
            transcendentals=0,
            bytes_accessed=2 * M * Cin + 2 * KKC * Cout + 2 * M * Cout,
        ),
    )(x_nhwc, w2d)

    inv_m = 1.0 / float(M)
    mean = jnp.sum(psum, axis=0) * inv_m
    var = jnp.maximum(jnp.sum(psq, axis=0) * inv_m - mean * mean, 0.0)
    g2d = gamma.reshape(1, Cout).astype(jnp.float32)
    b2d = beta.reshape(1, Cout).astype(jnp.float32)
    scale = g2d * jax.lax.rsqrt(var + eps)
    shift = b2d - mean * scale

    tm = 8192
    while M % tm:
        tm //= 2
    out2d = pl.pallas_call(
        _bn_relu_kernel,
        out_shape=jax.ShapeDtypeStruct((M, Cout), jnp.float32),
        grid=(M // tm,),
        in_specs=[
            pl.BlockSpec((tm, Cout), lambda i: (i, 0)),
            pl.BlockSpec((1, Cout), lambda i: (0, 0)),
            pl.BlockSpec((1, Cout), lambda i: (0, 0)),
        ],
        out_specs=pl.BlockSpec((tm, Cout), lambda i: (i, 0)),
        compiler_params=pltpu.CompilerParams(
            dimension_semantics=("parallel",),
            vmem_limit_bytes=_VMEM_LIMIT,
        ),
        cost_estimate=pl.CostEstimate(
            flops=3 * M * Cout,
            transcendentals=0,
            bytes_accessed=6 * M * Cout,
        ),
    )(y2d, scale, shift)

    return jnp.transpose(out2d.reshape(N, Ho, Wo, Cout), (0, 3, 1, 2))


# speedup vs baseline: 3.2642x; 1.0240x over previous
import functools

import jax
import jax.numpy as jnp
from jax.experimental import pallas as pl
from jax.experimental.pallas import tpu as pltpu

_VMEM_LIMIT = 100 * 1024 * 1024


def _conv_stats_kernel(x_ref, w_ref, y_ref, psum_ref, psq_ref, *, kk, ho, wo):
    pad = (kk - 1) // 2
    xs = jnp.pad(
        x_ref[...].astype(jnp.bfloat16),
        ((0, 0), (pad, pad), (pad, pad), (0, 0)),
    )
    nb = xs.shape[0]
    cols = [
        xs[:, kh:kh + ho, kw:kw + wo, :]
        for kh in range(kk) for kw in range(kk)
    ]
    p = jnp.concatenate(cols, axis=-1).reshape(nb * ho * wo, -1)
    yf = jnp.dot(p, w_ref[...], preferred_element_type=jnp.float32)
    y_ref[...] = yf.astype(y_ref.dtype)
    psum_ref[...] = jnp.sum(yf, axis=0, keepdims=True)[None]
    psq_ref[...] = jnp.sum(yf * yf, axis=0, keepdims=True)[None]


def _bn_relu_kernel(y_ref, scale_ref, shift_ref, o_ref):
    o_ref[...] = jnp.maximum(
        y_ref[...].astype(jnp.float32) * scale_ref[...] + shift_ref[...], 0.0
    )


@functools.partial(jax.jit, static_argnames=())
def kernel(x, w, b, gamma, beta):
    eps = 1e-5
    N, Cin, H, W = x.shape
    Cout = w.shape[0]
    K = w.shape[2]
    Ho, Wo = H, W
    HW = Ho * Wo
    M = N * HW
    KKC = K * K * Cin
    pad = (K - 1) // 2
    Hp, Wp = Ho + 2 * pad, Wo + 2 * pad
    del b

    x_nhwc = jnp.transpose(x, (0, 2, 3, 1))
    w2d = jnp.transpose(w, (2, 3, 1, 0)).reshape(KKC, Cout).astype(jnp.bfloat16)

    nb = 8
    while N % nb:
        nb //= 2
    G = N // nb
    body = functools.partial(_conv_stats_kernel, kk=K, ho=Ho, wo=Wo)
    y2d, psum, psq = pl.pallas_call(
        body,
        out_shape=(
            jax.ShapeDtypeStruct((M, Cout), jnp.bfloat16),
            jax.ShapeDtypeStruct((G, 1, Cout), jnp.float32),
            jax.ShapeDtypeStruct((G, 1, Cout), jnp.float32),
        ),
        grid=(G,),
        in_specs=[
            pl.BlockSpec((nb, Ho, Wo, Cin), lambda i: (i, 0, 0, 0)),
            pl.BlockSpec((KKC, Cout), lambda i: (0, 0)),
        ],
        out_specs=[
            pl.BlockSpec((nb * HW, Cout), lambda i: (i, 0)),
            pl.BlockSpec((1, 1, Cout), lambda i: (i, 0, 0)),
            pl.BlockSpec((1, 1, Cout), lambda i: (i, 0, 0)),
        ],
        compiler_params=pltpu.CompilerParams(
            dimension_semantics=("parallel",),
            vmem_limit_bytes=_VMEM_LIMIT,
        ),
        cost_estimate=pl.CostEstimate(
            flops=2 * M * KKC * Cout,
            transcendentals=0,
            bytes_accessed=2 * M * Cin + 2 * KKC * Cout + 2 * M * Cout,
        ),
    )(x_nhwc, w2d)

    inv_m = 1.0 / float(M)
    mean = jnp.sum(psum, axis=0) * inv_m
    var = jnp.maximum(jnp.sum(psq, axis=0) * inv_m - mean * mean, 0.0)
    g2d = gamma.reshape(1, Cout).astype(jnp.float32)
    b2d = beta.reshape(1, Cout).astype(jnp.float32)
    scale = g2d * jax.lax.rsqrt(var + eps)
    shift = b2d - mean * scale

    tm = 16384
    while M % tm:
        tm //= 2
    out2d = pl.pallas_call(
        _bn_relu_kernel,
        out_shape=jax.ShapeDtypeStruct((M, Cout), jnp.float32),
        grid=(M // tm,),
        in_specs=[
            pl.BlockSpec((tm, Cout), lambda i: (i, 0)),
            pl.BlockSpec((1, Cout), lambda i: (0, 0)),
            pl.BlockSpec((1, Cout), lambda i: (0, 0)),
        ],
        out_specs=pl.BlockSpec((tm, Cout), lambda i: (i, 0)),
        compiler_params=pltpu.CompilerParams(
            dimension_semantics=("parallel",),
            vmem_limit_bytes=_VMEM_LIMIT,
        ),
        cost_estimate=pl.CostEstimate(
            flops=3 * M * Cout,
            transcendentals=0,
            bytes_accessed=6 * M * Cout,
        ),
    )(y2d, scale, shift)

    return jnp.transpose(out2d.reshape(N, Ho, Wo, Cout), (0, 3, 1, 2))
